# baseline (speedup 1.0000x reference)
import functools

import jax
import jax.numpy as jnp
from jax.experimental import pallas as pl
from jax.experimental.pallas import tpu as pltpu

LAMBDA_MEM = 0.3


def _fused_kernel(x_ref, bank_ref, o_ref):
    xb = x_ref[0]
    c = xb.shape[0]
    hw = xb.shape[1]
    z = jnp.sum(xb, axis=1, keepdims=True) * (1.0 / hw)
    bank = bank_ref[...]
    logits = jax.lax.dot_general(
        bank, z, (((1,), (0,)), ((), ())),
        preferred_element_type=jnp.float32,
    ) * (c ** -0.5)
    logits = logits - jnp.max(logits)
    w = jnp.exp(logits)
    w = w * (1.0 / jnp.sum(w))
    m = jax.lax.dot_general(
        bank, w, (((0,), (0,)), ((), ())),
        preferred_element_type=jnp.float32,
    )
    o_ref[0] = xb + LAMBDA_MEM * m


@functools.partial(jax.jit, static_argnames=())
def kernel(x, memory_bank, centroid):
    del centroid
    B, C, H, W = x.shape
    x3 = x.reshape(B, C, H * W)
    out3 = pl.pallas_call(
        _fused_kernel,
        grid=(B,),
        in_specs=[
            pl.BlockSpec((1, C, H * W), lambda b: (b, 0, 0)),
            pl.BlockSpec(memory_bank.shape, lambda b: (0, 0)),
        ],
        out_specs=pl.BlockSpec((1, C, H * W), lambda b: (b, 0, 0)),
        out_shape=jax.ShapeDtypeStruct((B, C, H * W), x.dtype),
        compiler_params=pltpu.CompilerParams(
            dimension_semantics=("parallel",),
        ),
    )(x3, memory_bank)
    return out3.reshape(B, C, H, W)
